# baseline (speedup 1.0000x reference)
import functools

import jax
import jax.numpy as jnp
from jax.experimental import pallas as pl
from jax.experimental.pallas import tpu as pltpu

_SPATIAL_SCALE = 0.0625
_P = 7
_S = 4
_TRANS_STD = 0.1

_BN = 32
_BINS = _BN * _P * _P
_CB = 16


def _pool_kernel(rois_ref, off_ref, data_ref, g_ref, out_ref):
    f32 = jnp.float32
    H = W = 128
    nb = _BINS

    def col(r, i):
        return r[:, i:i + 1]

    rois = rois_ref[...].astype(f32)
    bidx = col(rois, 0)
    roi_sw = jnp.round(col(rois, 1)) * _SPATIAL_SCALE - 0.5
    roi_sh = jnp.round(col(rois, 2)) * _SPATIAL_SCALE - 0.5
    roi_ew = jnp.round(col(rois, 3) + 1.0) * _SPATIAL_SCALE - 0.5
    roi_eh = jnp.round(col(rois, 4) + 1.0) * _SPATIAL_SCALE - 0.5
    roi_w = jnp.maximum(roi_ew - roi_sw, 0.1)
    roi_h = jnp.maximum(roi_eh - roi_sh, 0.1)
    bin_w = roi_w / _P
    bin_h = roi_h / _P
    sub_w = bin_w / _S
    sub_h = bin_h / _S

    def per_bin(v):
        return jnp.broadcast_to(v[:, None, :], (_BN, _P * _P, 1)).reshape(nb, 1)

    loc = jax.lax.broadcasted_iota(jnp.int32, (nb, 1), 0) % (_P * _P)
    pf = (loc // _P).astype(f32)
    qf = (loc % _P).astype(f32)

    off = off_ref[...]
    tx = off[:, 0:1] * _TRANS_STD
    ty = off[:, 1:2] * _TRANS_STD

    wstart = qf * per_bin(bin_w) + per_bin(roi_sw) + tx * per_bin(roi_w)
    hstart = pf * per_bin(bin_h) + per_bin(roi_sh) + ty * per_bin(roi_h)
    sub_w_b = per_bin(sub_w)
    sub_h_b = per_bin(sub_h)
    bidx_b = per_bin(bidx)

    ylane = jax.lax.broadcasted_iota(jnp.int32, (nb, 2 * H), 1)
    yf = (ylane % H).astype(f32)
    bm = (ylane >= H) == (bidx_b > 0.5)

    xlane = jax.lax.broadcasted_iota(jnp.int32, (nb, W), 1)
    xf = xlane.astype(f32)

    def weights(start, sub, lane_f, size):
        acc = jnp.zeros(lane_f.shape, f32)
        cnt = jnp.zeros((nb, 1), f32)
        for s in range(_S):
            t = start + (s * 1.0) * sub
            m = ((t >= -0.5) & (t <= size - 0.5)).astype(f32)
            tc = jnp.clip(t, 0.0, size - 1.0)
            t0 = jnp.floor(tc)
            d = tc - t0
            t1 = jnp.ceil(tc)
            w_lo = (1.0 - d) * m
            w_hi = d * m
            acc = acc + jnp.where(lane_f == t0, w_lo, 0.0)
            acc = acc + jnp.where(lane_f == t1, w_hi, 0.0)
            cnt = cnt + m
        return acc, cnt

    a_w, cnt_h = weights(hstart, sub_h_b, yf, H)
    a_w = jnp.where(bm, a_w, 0.0)
    bx, cnt_w = weights(wstart, sub_w_b, xf, W)

    cnt = cnt_h * cnt_w
    rcp = jnp.where(cnt > 0.0, 1.0 / jnp.maximum(cnt, 1.0), 0.0)

    g = g_ref[...]
    a16 = a_w.astype(jnp.bfloat16)
    bx_rep = pltpu.repeat(bx.astype(jnp.bfloat16), _CB, axis=1)
    for ci in range(128 // _CB):
        dc = data_ref[:, ci * _CB * W:(ci + 1) * _CB * W]
        tmp = jnp.dot(a16, dc, preferred_element_type=f32)
        prod = tmp.astype(jnp.bfloat16) * bx_rep
        outc = jnp.dot(prod, g, preferred_element_type=f32)
        out_ref[:, ci * _CB:(ci + 1) * _CB] = outc * rcp


@jax.jit
def kernel(data, rois, offset):
    B, C, H, W = data.shape
    N = rois.shape[0]
    n_blocks = N // _BN

    data_t = data.transpose(0, 2, 1, 3).reshape(B * H, C * W).astype(jnp.bfloat16)
    off_b = offset.transpose(0, 2, 3, 1).reshape(N * _P * _P, 2)
    g = (jax.lax.broadcasted_iota(jnp.int32, (_CB * W, _CB), 0) // W
         == jax.lax.broadcasted_iota(jnp.int32, (_CB * W, _CB), 1)
         ).astype(jnp.bfloat16)

    out = pl.pallas_call(
        _pool_kernel,
        grid=(n_blocks,),
        in_specs=[
            pl.BlockSpec((_BN, 5), lambda i: (i, 0)),
            pl.BlockSpec((_BINS, 2), lambda i: (i, 0)),
            pl.BlockSpec((B * H, C * W), lambda i: (0, 0)),
            pl.BlockSpec((_CB * W, _CB), lambda i: (0, 0)),
        ],
        out_specs=pl.BlockSpec((_BINS, C), lambda i: (i, 0)),
        out_shape=jax.ShapeDtypeStruct((N * _P * _P, C), jnp.float32),
        compiler_params=pltpu.CompilerParams(
            dimension_semantics=("arbitrary",),
            vmem_limit_bytes=56 * 1024 * 1024,
        ),
    )(rois, off_b, data_t, g)

    return out.reshape(N, _P, _P, C).transpose(0, 3, 1, 2)


# speedup vs baseline: 1.2456x; 1.2456x over previous
import functools

import jax
import jax.numpy as jnp
from jax.experimental import pallas as pl
from jax.experimental.pallas import tpu as pltpu

_SPATIAL_SCALE = 0.0625
_P = 7
_S = 4
_TRANS_STD = 0.1

_BN = 16
_BINS = _BN * _P * _P
_CB = 32


def _pool_kernel(rois_ref, off_ref, data_ref, g_ref, out_ref):
    f32 = jnp.float32
    H = W = 128
    nb = _BINS

    def col(r, i):
        return r[:, i:i + 1]

    rois = rois_ref[...].astype(f32)
    bidx = col(rois, 0)
    roi_sw = jnp.round(col(rois, 1)) * _SPATIAL_SCALE - 0.5
    roi_sh = jnp.round(col(rois, 2)) * _SPATIAL_SCALE - 0.5
    roi_ew = jnp.round(col(rois, 3) + 1.0) * _SPATIAL_SCALE - 0.5
    roi_eh = jnp.round(col(rois, 4) + 1.0) * _SPATIAL_SCALE - 0.5
    roi_w = jnp.maximum(roi_ew - roi_sw, 0.1)
    roi_h = jnp.maximum(roi_eh - roi_sh, 0.1)
    bin_w = roi_w / _P
    bin_h = roi_h / _P
    sub_w = bin_w / _S
    sub_h = bin_h / _S

    def per_bin(v):
        return jnp.broadcast_to(v[:, None, :], (_BN, _P * _P, 1)).reshape(nb, 1)

    loc = jax.lax.broadcasted_iota(jnp.int32, (nb, 1), 0) % (_P * _P)
    pf = (loc // _P).astype(f32)
    qf = (loc % _P).astype(f32)

    off = off_ref[...]
    tx = off[:, 0:1] * _TRANS_STD
    ty = off[:, 1:2] * _TRANS_STD

    wstart = qf * per_bin(bin_w) + per_bin(roi_sw) + tx * per_bin(roi_w)
    hstart = pf * per_bin(bin_h) + per_bin(roi_sh) + ty * per_bin(roi_h)
    sub_w_b = per_bin(sub_w)
    sub_h_b = per_bin(sub_h)
    bidx_b = per_bin(bidx)

    ylane = jax.lax.broadcasted_iota(jnp.int32, (nb, 2 * H), 1)
    yf = (ylane % H).astype(f32)
    bm = (ylane >= H) == (bidx_b > 0.5)

    xlane = jax.lax.broadcasted_iota(jnp.int32, (nb, W), 1)
    xf = xlane.astype(f32)

    def weights(start, sub, lane_f, size):
        acc = jnp.zeros(lane_f.shape, f32)
        cnt = jnp.zeros((nb, 1), f32)
        for s in range(_S):
            t = start + (s * 1.0) * sub
            m = ((t >= -0.5) & (t <= size - 0.5)).astype(f32)
            tc = jnp.clip(t, 0.0, size - 1.0)
            t0 = jnp.floor(tc)
            d = tc - t0
            t1 = jnp.ceil(tc)
            w_lo = (1.0 - d) * m
            w_hi = d * m
            acc = acc + jnp.where(lane_f == t0, w_lo, 0.0)
            acc = acc + jnp.where(lane_f == t1, w_hi, 0.0)
            cnt = cnt + m
        return acc, cnt

    a_w, cnt_h = weights(hstart, sub_h_b, yf, H)
    a_w = jnp.where(bm, a_w, 0.0)
    bx, cnt_w = weights(wstart, sub_w_b, xf, W)

    cnt = cnt_h * cnt_w
    rcp = jnp.where(cnt > 0.0, 1.0 / jnp.maximum(cnt, 1.0), 0.0)

    g = g_ref[...]
    a16 = a_w.astype(jnp.bfloat16)
    bx_rep = pltpu.repeat(bx.astype(jnp.bfloat16), _CB, axis=1)
    for ci in range(128 // _CB):
        dc = data_ref[:, ci * _CB * W:(ci + 1) * _CB * W]
        tmp = jnp.dot(a16, dc, preferred_element_type=f32)
        prod = tmp.astype(jnp.bfloat16) * bx_rep
        outc = jnp.dot(prod, g, preferred_element_type=f32)
        out_ref[:, ci * _CB:(ci + 1) * _CB] = outc * rcp


@jax.jit
def kernel(data, rois, offset):
    B, C, H, W = data.shape
    N = rois.shape[0]
    n_blocks = N // _BN

    data_t = data.transpose(0, 2, 1, 3).reshape(B * H, C * W).astype(jnp.bfloat16)
    off_b = offset.transpose(0, 2, 3, 1).reshape(N * _P * _P, 2)
    g = (jax.lax.broadcasted_iota(jnp.int32, (_CB * W, _CB), 0) // W
         == jax.lax.broadcasted_iota(jnp.int32, (_CB * W, _CB), 1)
         ).astype(jnp.bfloat16)

    out = pl.pallas_call(
        _pool_kernel,
        grid=(n_blocks,),
        in_specs=[
            pl.BlockSpec((_BN, 5), lambda i: (i, 0)),
            pl.BlockSpec((_BINS, 2), lambda i: (i, 0)),
            pl.BlockSpec((B * H, C * W), lambda i: (0, 0)),
            pl.BlockSpec((_CB * W, _CB), lambda i: (0, 0)),
        ],
        out_specs=pl.BlockSpec((_BINS, C), lambda i: (i, 0)),
        out_shape=jax.ShapeDtypeStruct((N * _P * _P, C), jnp.float32),
        compiler_params=pltpu.CompilerParams(
            dimension_semantics=("arbitrary",),
            vmem_limit_bytes=56 * 1024 * 1024,
        ),
    )(rois, off_b, data_t, g)

    return out.reshape(N, _P, _P, C).transpose(0, 3, 1, 2)
